# baseline (speedup 1.0000x reference)
import jax
import jax.numpy as jnp
from jax import lax
from jax.experimental import pallas as pl
from jax.experimental.pallas import tpu as pltpu
from jax.experimental.pallas import tpu_sc as plsc

NC = 2
NS = 16
NW = NC * NS

B, L, W = 4096, 200, 32
BPW = B // NW
LB = 5
NCH = L // LB
TROWS = 2056
PAD_ROW = 2048


def _tc_select_t_body(x_ref, m_ref, o_ref):
    sel = jnp.where(m_ref[...], x_ref[...], PAD_ROW)
    o_ref[...] = sel.T


def _masked_indices_t(idx, mask):
    return pl.pallas_call(
        _tc_select_t_body,
        grid=(B // 128, 2),
        in_specs=[
            pl.BlockSpec((128, 128), lambda i, j: (i, j)),
            pl.BlockSpec((128, 128), lambda i, j: (i, j)),
        ],
        out_specs=pl.BlockSpec((128, 128), lambda i, j: (j, i)),
        out_shape=jax.ShapeDtypeStruct((L, B), jnp.int32),
    )(idx, mask)


def _sc_body(idxt_hbm, table_hbm, out_hbm,
             table_v, in0, in1, out0, out1,
             insem0, insem1, outsem0, outsem1):
    wid = lax.axis_index("s") * NC + lax.axis_index("c")
    b0 = wid * BPW
    in_bufs = (in0, in1)
    out_bufs = (out0, out1)
    insems = (insem0, insem1)
    outsems = (outsem0, outsem1)

    def start_in(c, slot):
        pltpu.async_copy(idxt_hbm.at[pl.ds(c * LB, LB), pl.ds(b0, BPW)],
                         in_bufs[slot], insems[slot])

    def out_slice(c):
        return out_hbm.at[pl.ds(c * LB, LB), :, wid, :, :]

    start_in(0, 0)
    start_in(1, 1)
    pltpu.sync_copy(table_hbm, table_v)

    def do_chunk(c, slot):
        in_v = in_bufs[slot]
        out_v = out_bufs[slot]
        pltpu.make_async_copy(idxt_hbm.at[pl.ds(0, LB), pl.ds(b0, BPW)],
                              in_v, insems[slot]).wait()

        @pl.when(c >= 2)
        def _():
            pltpu.make_async_copy(out_v, out_slice(0), outsems[slot]).wait()

        def group(g, carry):
            sl = pl.ds(g * 16, 16)
            for ll in range(LB):
                base = in_v[ll, sl] * W
                for w in range(W):
                    out_v[ll, w // 8, w % 8, sl] = plsc.load_gather(
                        table_v, [base + w])
            return carry

        lax.fori_loop(0, BPW // 16, group, 0)

        @pl.when(c + 2 < NCH)
        def _():
            start_in(c + 2, slot)

        pltpu.async_copy(out_v, out_slice(c), outsems[slot])

    def body(t, carry):
        do_chunk(2 * t, 0)
        do_chunk(2 * t + 1, 1)
        return carry

    lax.fori_loop(0, NCH // 2, body, 0)
    pltpu.make_async_copy(out0, out_slice(0), outsems[0]).wait()
    pltpu.make_async_copy(out1, out_slice(0), outsems[1]).wait()


def _sc_gather_t(idxt, table_flat):
    mesh = plsc.VectorSubcoreMesh(core_axis_name="c", subcore_axis_name="s")
    kfn = pl.kernel(
        _sc_body,
        out_type=jax.ShapeDtypeStruct((L, W // 8, NW, 8, B // NW // 8 * 8),
                                      jnp.float32),
        mesh=mesh,
        scratch_types=[
            pltpu.VMEM((TROWS * W,), jnp.float32),
            pltpu.VMEM((LB, BPW), jnp.int32),
            pltpu.VMEM((LB, BPW), jnp.int32),
            pltpu.VMEM((LB, W // 8, 8, BPW), jnp.float32),
            pltpu.VMEM((LB, W // 8, 8, BPW), jnp.float32),
            pltpu.SemaphoreType.DMA,
            pltpu.SemaphoreType.DMA,
            pltpu.SemaphoreType.DMA,
            pltpu.SemaphoreType.DMA,
        ],
        compiler_params=pltpu.CompilerParams(use_tc_tiling_on_sc=False,
                                             needs_layout_passes=False),
    )
    return kfn(idxt, table_flat)


@jax.jit
def _run(idx, mask, table_flat):
    idxt = _masked_indices_t(idx, mask)
    return _sc_gather_t(idxt, table_flat)


def kernel(obs_pos, obs_mask, embedding_table):
    table_flat = jnp.concatenate(
        [embedding_table, jnp.zeros((TROWS - 2048, W), jnp.float32)],
        axis=0).reshape(-1)
    out5 = _run(obs_pos.astype(jnp.int32), obs_mask, table_flat)
    return out5.transpose(2, 4, 0, 1, 3).reshape(B, L, W)


# speedup vs baseline: 1.5891x; 1.5891x over previous
import jax
import jax.numpy as jnp
from jax import lax
from jax.experimental import pallas as pl
from jax.experimental.pallas import tpu as pltpu
from jax.experimental.pallas import tpu_sc as plsc

NC = 2
NS = 16
NW = NC * NS

B, L, W = 4096, 200, 32
BPW = B // NW
LB = 5
NCH = L // LB
TROWS = 2056
PAD_ROW = 2048


def _tc_select_t_body(x_ref, m_ref, o_ref):
    sel = jnp.where(m_ref[...], x_ref[...], PAD_ROW)
    o_ref[...] = sel.T


def _masked_indices_t(idx, mask):
    return pl.pallas_call(
        _tc_select_t_body,
        grid=(B // 128, 2),
        in_specs=[
            pl.BlockSpec((128, 128), lambda i, j: (i, j)),
            pl.BlockSpec((128, 128), lambda i, j: (i, j)),
        ],
        out_specs=pl.BlockSpec((128, 128), lambda i, j: (j, i)),
        out_shape=jax.ShapeDtypeStruct((L, B), jnp.int32),
    )(idx, mask)


def _sc_body(idxt_hbm, table_hbm, out_hbm,
             table_v, in0, in1, out0, out1,
             insem0, insem1, outsem0, outsem1):
    wid = lax.axis_index("s") * NC + lax.axis_index("c")
    b0 = wid * BPW
    in_bufs = (in0, in1)
    out_bufs = (out0, out1)
    insems = (insem0, insem1)
    outsems = (outsem0, outsem1)

    def start_in(c, slot):
        pltpu.async_copy(idxt_hbm.at[pl.ds(c * LB, LB), pl.ds(b0, BPW)],
                         in_bufs[slot], insems[slot])

    def out_slice(c):
        return out_hbm.at[pl.ds(c * LB, LB), :, wid, :, :]

    start_in(0, 0)
    start_in(1, 1)
    pltpu.sync_copy(table_hbm, table_v)

    def do_chunk(c, slot):
        in_v = in_bufs[slot]
        out_v = out_bufs[slot]
        pltpu.make_async_copy(idxt_hbm.at[pl.ds(0, LB), pl.ds(b0, BPW)],
                              in_v, insems[slot]).wait()

        @pl.when(c >= 2)
        def _():
            pltpu.make_async_copy(out_v, out_slice(0), outsems[slot]).wait()

        @plsc.parallel_loop(0, BPW, step=16)
        def _(b):
            sl = pl.ds(b, 16)
            for ll in range(LB):
                base = in_v[ll, sl] * W
                for w in range(W):
                    out_v[ll, w // 8, w % 8, sl] = plsc.load_gather(
                        table_v, [base + w])

        @pl.when(c + 2 < NCH)
        def _():
            start_in(c + 2, slot)

        pltpu.async_copy(out_v, out_slice(c), outsems[slot])

    def body(t, carry):
        do_chunk(2 * t, 0)
        do_chunk(2 * t + 1, 1)
        return carry

    lax.fori_loop(0, NCH // 2, body, 0)
    pltpu.make_async_copy(out0, out_slice(0), outsems[0]).wait()
    pltpu.make_async_copy(out1, out_slice(0), outsems[1]).wait()


def _sc_gather_t(idxt, table_flat):
    mesh = plsc.VectorSubcoreMesh(core_axis_name="c", subcore_axis_name="s")
    kfn = pl.kernel(
        _sc_body,
        out_type=jax.ShapeDtypeStruct((L, W // 8, NW, 8, B // NW // 8 * 8),
                                      jnp.float32),
        mesh=mesh,
        scratch_types=[
            pltpu.VMEM((TROWS * W,), jnp.float32),
            pltpu.VMEM((LB, BPW), jnp.int32),
            pltpu.VMEM((LB, BPW), jnp.int32),
            pltpu.VMEM((LB, W // 8, 8, BPW), jnp.float32),
            pltpu.VMEM((LB, W // 8, 8, BPW), jnp.float32),
            pltpu.SemaphoreType.DMA,
            pltpu.SemaphoreType.DMA,
            pltpu.SemaphoreType.DMA,
            pltpu.SemaphoreType.DMA,
        ],
        compiler_params=pltpu.CompilerParams(use_tc_tiling_on_sc=False,
                                             needs_layout_passes=False),
    )
    return kfn(idxt, table_flat)


@jax.jit
def _run(idx, mask, table_flat):
    idxt = _masked_indices_t(idx, mask)
    return _sc_gather_t(idxt, table_flat)


def kernel(obs_pos, obs_mask, embedding_table):
    table_flat = jnp.concatenate(
        [embedding_table, jnp.zeros((TROWS - 2048, W), jnp.float32)],
        axis=0).reshape(-1)
    out5 = _run(obs_pos.astype(jnp.int32), obs_mask, table_flat)
    return out5.transpose(2, 4, 0, 1, 3).reshape(B, L, W)


# speedup vs baseline: 2.9415x; 1.8511x over previous
import jax
import jax.numpy as jnp
from jax import lax
from jax.experimental import pallas as pl
from jax.experimental.pallas import tpu as pltpu
from jax.experimental.pallas import tpu_sc as plsc

NC = 2
NS = 16
NW = NC * NS

B, L, W = 4096, 200, 32
BPW = B // NW
LB = 5
NCH = L // LB
TROWS = 2056
PAD_ROW = 2048
TSTRIDE = W + 1


def _tc_select_t_body(x_ref, m_ref, o_ref):
    sel = jnp.where(m_ref[...], x_ref[...], PAD_ROW)
    o_ref[...] = sel.T


def _masked_indices_t(idx, mask):
    return pl.pallas_call(
        _tc_select_t_body,
        grid=(B // 128, 2),
        in_specs=[
            pl.BlockSpec((128, 128), lambda i, j: (i, j)),
            pl.BlockSpec((128, 128), lambda i, j: (i, j)),
        ],
        out_specs=pl.BlockSpec((128, 128), lambda i, j: (j, i)),
        out_shape=jax.ShapeDtypeStruct((L, B), jnp.int32),
    )(idx, mask)


def _sc_body(idxt_hbm, table_hbm, out_hbm,
             table_v, in0, in1, out0, out1,
             insem0, insem1, outsem0, outsem1):
    wid = lax.axis_index("s") * NC + lax.axis_index("c")
    b0 = wid * BPW
    in_bufs = (in0, in1)
    out_bufs = (out0, out1)
    insems = (insem0, insem1)
    outsems = (outsem0, outsem1)

    def start_in(c, slot):
        pltpu.async_copy(idxt_hbm.at[pl.ds(c * LB, LB), pl.ds(b0, BPW)],
                         in_bufs[slot], insems[slot])

    def out_slice(c):
        return out_hbm.at[pl.ds(c * LB, LB), :, wid, :, :]

    start_in(0, 0)
    start_in(1, 1)
    pltpu.sync_copy(table_hbm, table_v)

    def do_chunk(c, slot):
        in_v = in_bufs[slot]
        out_v = out_bufs[slot]
        pltpu.make_async_copy(idxt_hbm.at[pl.ds(0, LB), pl.ds(b0, BPW)],
                              in_v, insems[slot]).wait()

        @pl.when(c >= 2)
        def _():
            pltpu.make_async_copy(out_v, out_slice(0), outsems[slot]).wait()

        @plsc.parallel_loop(0, BPW, step=16)
        def _(b):
            sl = pl.ds(b, 16)
            for ll in range(LB):
                base = in_v[ll, sl] * TSTRIDE
                for w in range(W):
                    out_v[ll, w // 8, w % 8, sl] = plsc.load_gather(
                        table_v, [base + w])

        @pl.when(c + 2 < NCH)
        def _():
            start_in(c + 2, slot)

        pltpu.async_copy(out_v, out_slice(c), outsems[slot])

    def body(t, carry):
        do_chunk(2 * t, 0)
        do_chunk(2 * t + 1, 1)
        return carry

    lax.fori_loop(0, NCH // 2, body, 0)
    pltpu.make_async_copy(out0, out_slice(0), outsems[0]).wait()
    pltpu.make_async_copy(out1, out_slice(0), outsems[1]).wait()


def _sc_gather_t(idxt, table_flat):
    mesh = plsc.VectorSubcoreMesh(core_axis_name="c", subcore_axis_name="s")
    kfn = pl.kernel(
        _sc_body,
        out_type=jax.ShapeDtypeStruct((L, W // 8, NW, 8, B // NW // 8 * 8),
                                      jnp.float32),
        mesh=mesh,
        scratch_types=[
            pltpu.VMEM((TROWS * TSTRIDE,), jnp.float32),
            pltpu.VMEM((LB, BPW), jnp.int32),
            pltpu.VMEM((LB, BPW), jnp.int32),
            pltpu.VMEM((LB, W // 8, 8, BPW), jnp.float32),
            pltpu.VMEM((LB, W // 8, 8, BPW), jnp.float32),
            pltpu.SemaphoreType.DMA,
            pltpu.SemaphoreType.DMA,
            pltpu.SemaphoreType.DMA,
            pltpu.SemaphoreType.DMA,
        ],
        compiler_params=pltpu.CompilerParams(use_tc_tiling_on_sc=False,
                                             needs_layout_passes=False),
    )
    return kfn(idxt, table_flat)


@jax.jit
def _run(idx, mask, table_flat):
    idxt = _masked_indices_t(idx, mask)
    return _sc_gather_t(idxt, table_flat)


def kernel(obs_pos, obs_mask, embedding_table):
    table_flat = jnp.pad(embedding_table,
                         ((0, TROWS - 2048), (0, TSTRIDE - W))).reshape(-1)
    out5 = _run(obs_pos.astype(jnp.int32), obs_mask, table_flat)
    return out5.transpose(2, 4, 0, 1, 3).reshape(B, L, W)
